# baseline (speedup 1.0000x reference)
import functools

import jax
import jax.numpy as jnp
from jax import lax
from jax.experimental import pallas as pl
from jax.experimental.pallas import tpu as pltpu
from jax.experimental.pallas import tpu_sc as plsc

V = 1000
E = 64
NC = 2
NS = 16
NW = NC * NS
CHUNK = 64


def _fused_table(emb, w, b2d):
    def mm(e_ref, w_ref, b_ref, o_ref):
        o_ref[...] = lax.dot_general(
            e_ref[...], w_ref[...],
            dimension_numbers=(((1,), (1,)), ((), ())),
            preferred_element_type=jnp.float32,
        ) + b_ref[...]

    return pl.pallas_call(
        mm,
        out_shape=jax.ShapeDtypeStruct((V, V), jnp.float32),
    )(emb, w, b2d)


def _make_gather(n_tokens):
    per_w = n_tokens // NW
    n_chunks = per_w // CHUNK
    mesh = plsc.VectorSubcoreMesh(core_axis_name="c", subcore_axis_name="s")

    @functools.partial(
        pl.kernel, mesh=mesh,
        compiler_params=pltpu.CompilerParams(use_tc_tiling_on_sc=False),
        out_type=jax.ShapeDtypeStruct((n_tokens, V), jnp.float32),
        scratch_types=[
            pltpu.VMEM((n_chunks, CHUNK), jnp.int32),
            pltpu.VMEM((CHUNK, V), jnp.float32),
            pltpu.SemaphoreType.DMA,
        ],
    )
    def gather_k(p_hbm, idx_hbm, out_hbm, idx_v, rows_v, sem):
        wid = lax.axis_index("s") * NC + lax.axis_index("c")
        base = wid * per_w
        pltpu.sync_copy(idx_hbm.at[wid], idx_v)

        def body(c, carry):
            pltpu.async_copy(p_hbm.at[idx_v.at[c]], rows_v, sem).wait()
            pltpu.sync_copy(rows_v, out_hbm.at[pl.ds(base + c * CHUNK, CHUNK)])
            return carry

        lax.fori_loop(0, n_chunks, body, 0)

    return gather_k


def kernel(input_ids, embedding_table, linear_w, linear_b):
    bsz, seq = input_ids.shape
    n_tokens = bsz * seq
    p = _fused_table(embedding_table, linear_w, linear_b.reshape(1, V))
    ids = input_ids.astype(jnp.int32).reshape(NW, n_tokens // (NW * CHUNK), CHUNK)
    out = _make_gather(n_tokens)(p, ids)
    return out.reshape(bsz, seq, V)


# speedup vs baseline: 1.7369x; 1.7369x over previous
import functools

import jax
import jax.numpy as jnp
from jax import lax
from jax.experimental import pallas as pl
from jax.experimental.pallas import tpu as pltpu
from jax.experimental.pallas import tpu_sc as plsc

V = 1000
E = 64
VP = 1024
NCB = 8
TAIL = V - 7 * 128
NC = 2
NS = 16
NW = NC * NS


def _fused_table(emb, w_pad, b2):
    def mm(e_ref, w_ref, b_ref, o_ref):
        o_ref[0] = lax.dot_general(
            e_ref[...], w_ref[...],
            dimension_numbers=(((1,), (1,)), ((), ())),
            preferred_element_type=jnp.float32,
        ) + b_ref[0]

    return pl.pallas_call(
        mm,
        grid=(NCB,),
        in_specs=[
            pl.BlockSpec((V, E), lambda cb: (0, 0)),
            pl.BlockSpec((128, E), lambda cb: (cb, 0)),
            pl.BlockSpec((1, 1, 128), lambda cb: (cb, 0, 0)),
        ],
        out_specs=pl.BlockSpec((1, V, 128), lambda cb: (cb, 0, 0)),
        out_shape=jax.ShapeDtypeStruct((NCB, V, 128), jnp.float32),
    )(emb, w_pad, b2)


def _make_gather(bsz, seq):
    n_per_w = bsz // NW
    mesh = plsc.VectorSubcoreMesh(core_axis_name="c", subcore_axis_name="s")

    @functools.partial(
        pl.kernel, mesh=mesh,
        out_type=jax.ShapeDtypeStruct((bsz, seq, V), jnp.float32),
        scratch_types=[
            pltpu.VMEM((n_per_w, seq), jnp.int32),
            pltpu.VMEM((NCB, seq, 128), jnp.float32),
            pltpu.VMEM((NCB, seq, 128), jnp.float32),
            pltpu.SemaphoreType.DMA,
            pltpu.SemaphoreType.DMA,
            pltpu.SemaphoreType.DMA,
            pltpu.SemaphoreType.DMA,
        ],
    )
    def gather_k(t_hbm, ids_hbm, out_hbm, idx_v, rows_a, rows_b,
                 sem_ga, sem_gb, sem_wa, sem_wb):
        wid = lax.axis_index("s") * NC + lax.axis_index("c")
        base = wid * n_per_w
        pltpu.sync_copy(ids_hbm.at[pl.ds(base, n_per_w)], idx_v)

        def issue_gathers(n_local, rows, sem):
            for cb in range(NCB):
                pltpu.async_copy(
                    t_hbm.at[cb].at[idx_v.at[n_local]], rows.at[cb], sem)

        def wait_gathers(n_local, rows, sem):
            for cb in range(NCB):
                pltpu.make_async_copy(
                    t_hbm.at[cb].at[idx_v.at[n_local]], rows.at[cb],
                    sem).wait()

        def tail_pieces(n_local, rows):
            b = base + n_local
            for l in range(seq):
                yield (rows.at[NCB - 1, l, pl.ds(0, TAIL)],
                       out_hbm.at[b, l, pl.ds((NCB - 1) * 128, TAIL)])

        def issue_writes(n_local, rows, sem):
            b = base + n_local
            for cb in range(NCB - 1):
                pltpu.async_copy(
                    rows.at[cb], out_hbm.at[b, :, pl.ds(cb * 128, 128)], sem)
            for src, dst in tail_pieces(n_local, rows):
                pltpu.async_copy(src, dst, sem)

        def wait_writes(n_local, rows, sem):
            b = base + n_local
            for cb in range(NCB - 1):
                pltpu.make_async_copy(
                    rows.at[cb], out_hbm.at[b, :, pl.ds(cb * 128, 128)],
                    sem).wait()
            for src, dst in tail_pieces(n_local, rows):
                pltpu.make_async_copy(src, dst, sem).wait()

        issue_gathers(0, rows_a, sem_ga)
        issue_gathers(1, rows_b, sem_gb)

        def body(j, carry):
            n0 = 2 * j
            n1 = n0 + 1
            wait_gathers(n0, rows_a, sem_ga)
            issue_writes(n0, rows_a, sem_wa)
            wait_writes(n0, rows_a, sem_wa)

            @pl.when(j + 1 < n_per_w // 2)
            def _():
                issue_gathers(n0 + 2, rows_a, sem_ga)

            wait_gathers(n1, rows_b, sem_gb)
            issue_writes(n1, rows_b, sem_wb)
            wait_writes(n1, rows_b, sem_wb)

            @pl.when(j + 1 < n_per_w // 2)
            def _():
                issue_gathers(n1 + 2, rows_b, sem_gb)

            return carry

        lax.fori_loop(0, n_per_w // 2, body, 0)

    return gather_k


def kernel(input_ids, embedding_table, linear_w, linear_b):
    bsz, seq = input_ids.shape
    w_pad = jnp.pad(linear_w, ((0, VP - V), (0, 0)))
    b2 = jnp.pad(linear_b, (0, VP - V)).reshape(NCB, 1, 128)
    t3 = _fused_table(embedding_table, w_pad, b2)
    return _make_gather(bsz, seq)(t3, input_ids.astype(jnp.int32))
